# baseline (speedup 1.0000x reference)
import functools

import jax
import jax.numpy as jnp
from jax import lax
from jax.experimental import pallas as pl
from jax.experimental.pallas import tpu as pltpu
from jax.experimental.pallas import tpu_sc as plsc

_BATCH = 16384
_FIELDS = 26
_DIM = 64
_BT = _BATCH * _FIELDS

_NC = 2
_NS = 16
_NW = _NC * _NS
_BPW = _BT // _NW
_CHUNK = 128
_NCH = _BPW // _CHUNK
_NBUF = 4


def _embed_body(tbl_hbm, idx_hbm, out_hbm, idx_v, rows_v, gsem):
    wid = lax.axis_index("s") * _NC + lax.axis_index("c")
    base = wid * _BPW

    pltpu.sync_copy(idx_hbm.at[wid], idx_v)

    def start_gather(chunk, slot):
        pltpu.make_async_copy(
            tbl_hbm.at[idx_v.at[chunk]], rows_v.at[slot], gsem.at[slot]
        ).start()

    for b in range(_NBUF):
        start_gather(b, b)

    def outer(j0):
        for b in range(_NBUF):
            chunk = j0 + b
            pltpu.make_async_copy(
                tbl_hbm.at[idx_v.at[chunk]], rows_v.at[b], gsem.at[b]
            ).wait()
            pltpu.sync_copy(
                rows_v.at[b], out_hbm.at[pl.ds(base + chunk * _CHUNK, _CHUNK)]
            )

            @pl.when(chunk + _NBUF < _NCH)
            def _():
                start_gather(chunk + _NBUF, b)

    pl.loop(0, _NCH, step=_NBUF)(outer)


@functools.partial(
    pl.kernel,
    mesh=plsc.VectorSubcoreMesh(core_axis_name="c", subcore_axis_name="s"),
    out_type=jax.ShapeDtypeStruct((_BT, _DIM), jnp.float32),
    scratch_types=[
        pltpu.VMEM((_NCH, _CHUNK), jnp.int32),
        pltpu.VMEM((_NBUF, _CHUNK, _DIM), jnp.float32),
        pltpu.SemaphoreType.DMA((_NBUF,)),
    ],
    compiler_params=pltpu.CompilerParams(use_tc_tiling_on_sc=False),
)
def _embed_call(tbl_hbm, idx_hbm, out_hbm, idx_v, rows_v, gsem):
    _embed_body(tbl_hbm, idx_hbm, out_hbm, idx_v, rows_v, gsem)


def kernel(input, weight):
    idx = input.reshape(_NW, _NCH, _CHUNK).astype(jnp.int32)
    out = _embed_call(weight, idx)
    return out.reshape(_BATCH, _FIELDS, _DIM)
